# baseline (speedup 1.0000x reference)
import functools

import jax
import jax.numpy as jnp
from jax.experimental import pallas as pl

BQ = 512
BR = 2048
BR2 = 4096


def _knn_mlp0(x1_ref, x2_ref, p1_ref, p2_ref, w0a_ref, w0b_ref, b0_ref,
              y0_ref, s0_ref, ss0_ref):
    b = pl.program_id(0)
    i = pl.program_id(1)
    x1 = x1_ref[0]
    x2 = x2_ref[0]
    rowsq = jnp.sum(x1 * x1, axis=1, keepdims=True)
    colsq = jnp.sum(x2 * x2, axis=0, keepdims=True)
    h = jnp.dot(x1, x2, preferred_element_type=jnp.float32)
    t = (0.5 * rowsq + 0.5 * colsq) - h

    inf = jnp.float32(jnp.inf)
    m0 = jnp.min(t, axis=1, keepdims=True)
    m1 = jnp.min(jnp.where(t == m0, inf, t), axis=1, keepdims=True)
    m2 = jnp.min(jnp.where(t == m0, inf, jnp.where(t == m1, inf, t)),
                 axis=1, keepdims=True)

    dists = [jnp.maximum(2.0 * m, 1e-10) for m in (m0, m1, m2)]
    w = [1.0 / dd for dd in dists]
    norm = jnp.maximum(w[0] + w[1] + w[2], 1e-8)
    wn = [wi / norm for wi in w]

    sel = jnp.where(t == m0, wn[0],
                    jnp.where(t == m1, wn[1],
                              jnp.where(t == m2, wn[2], 0.0)))
    interp = jnp.dot(sel, p2_ref[0], preferred_element_type=jnp.float32)

    y0 = (jnp.dot(p1_ref[0], w0a_ref[...], preferred_element_type=jnp.float32)
          + jnp.dot(interp, w0b_ref[...], preferred_element_type=jnp.float32)
          + b0_ref[...])
    y0_ref[0] = y0

    @pl.when(jnp.logical_and(b == 0, i == 0))
    def _():
        s0_ref[...] = jnp.zeros_like(s0_ref)
        ss0_ref[...] = jnp.zeros_like(ss0_ref)

    s0_ref[...] += jnp.sum(y0, axis=0, keepdims=True)
    ss0_ref[...] += jnp.sum(y0 * y0, axis=0, keepdims=True)


def _mlp1(y0_ref, s0_ref, ss0_ref, g0_ref, beta0_ref, w1t_ref, b1_ref,
          y1_ref, s1_ref, ss1_ref, *, n_total):
    i = pl.program_id(0)
    mean = s0_ref[...] / n_total
    var = ss0_ref[...] / n_total - mean * mean
    scale = g0_ref[...] * jax.lax.rsqrt(var + 1e-5)
    shift = beta0_ref[...] - mean * scale
    z = jnp.maximum(y0_ref[...] * scale + shift, 0.0)
    y1 = jnp.dot(z, w1t_ref[...], preferred_element_type=jnp.float32) + b1_ref[...]
    y1_ref[...] = y1

    @pl.when(i == 0)
    def _():
        s1_ref[...] = jnp.zeros_like(s1_ref)
        ss1_ref[...] = jnp.zeros_like(ss1_ref)

    s1_ref[...] += jnp.sum(y1, axis=0, keepdims=True)
    ss1_ref[...] += jnp.sum(y1 * y1, axis=0, keepdims=True)


def _bn2(y1_ref, s1_ref, ss1_ref, g1_ref, beta1_ref, out_ref, *, n_total):
    mean = s1_ref[...] / n_total
    var = ss1_ref[...] / n_total - mean * mean
    scale = g1_ref[...] * jax.lax.rsqrt(var + 1e-5)
    shift = beta1_ref[...] - mean * scale
    out_ref[...] = jnp.maximum(y1_ref[...] * scale + shift, 0.0)


def kernel(xyz1, xyz2, points1, points2, W0, b0, g0, beta0, W1, b1, g1, beta1):
    B, N1, _ = xyz1.shape
    N2 = xyz2.shape[1]
    C1 = points1.shape[2]
    C2 = points2.shape[2]
    CH0 = W0.shape[0]
    CH1 = W1.shape[0]
    f32 = jnp.float32

    x1p = jnp.pad(xyz1, ((0, 0), (0, 0), (0, 5)))
    x2p = jnp.pad(xyz2, ((0, 0), (0, 0), (0, 5))).transpose(0, 2, 1)
    w0a = W0[:, :C1].T
    w0b = W0[:, C1:].T
    w1t = W1.T

    grid_a = (B, N1 // BQ)
    y0, s0, ss0 = pl.pallas_call(
        _knn_mlp0,
        grid=grid_a,
        in_specs=[
            pl.BlockSpec((1, BQ, 8), lambda b, i: (b, i, 0)),
            pl.BlockSpec((1, 8, N2), lambda b, i: (b, 0, 0)),
            pl.BlockSpec((1, BQ, C1), lambda b, i: (b, i, 0)),
            pl.BlockSpec((1, N2, C2), lambda b, i: (b, 0, 0)),
            pl.BlockSpec((C1, CH0), lambda b, i: (0, 0)),
            pl.BlockSpec((C2, CH0), lambda b, i: (0, 0)),
            pl.BlockSpec((1, CH0), lambda b, i: (0, 0)),
        ],
        out_specs=[
            pl.BlockSpec((1, BQ, CH0), lambda b, i: (b, i, 0)),
            pl.BlockSpec((1, CH0), lambda b, i: (0, 0)),
            pl.BlockSpec((1, CH0), lambda b, i: (0, 0)),
        ],
        out_shape=[
            jax.ShapeDtypeStruct((B, N1, CH0), f32),
            jax.ShapeDtypeStruct((1, CH0), f32),
            jax.ShapeDtypeStruct((1, CH0), f32),
        ],
    )(x1p, x2p, points1, points2, w0a, w0b, b0.reshape(1, CH0))

    n_total = B * N1
    y0f = y0.reshape(B * N1, CH0)
    y1, s1, ss1 = pl.pallas_call(
        functools.partial(_mlp1, n_total=float(n_total)),
        grid=(B * N1 // BR,),
        in_specs=[
            pl.BlockSpec((BR, CH0), lambda i: (i, 0)),
            pl.BlockSpec((1, CH0), lambda i: (0, 0)),
            pl.BlockSpec((1, CH0), lambda i: (0, 0)),
            pl.BlockSpec((1, CH0), lambda i: (0, 0)),
            pl.BlockSpec((1, CH0), lambda i: (0, 0)),
            pl.BlockSpec((CH0, CH1), lambda i: (0, 0)),
            pl.BlockSpec((1, CH1), lambda i: (0, 0)),
        ],
        out_specs=[
            pl.BlockSpec((BR, CH1), lambda i: (i, 0)),
            pl.BlockSpec((1, CH1), lambda i: (0, 0)),
            pl.BlockSpec((1, CH1), lambda i: (0, 0)),
        ],
        out_shape=[
            jax.ShapeDtypeStruct((B * N1, CH1), f32),
            jax.ShapeDtypeStruct((1, CH1), f32),
            jax.ShapeDtypeStruct((1, CH1), f32),
        ],
    )(y0f, s0, ss0, g0.reshape(1, CH0), beta0.reshape(1, CH0), w1t,
      b1.reshape(1, CH1))

    out = pl.pallas_call(
        functools.partial(_bn2, n_total=float(n_total)),
        grid=(B * N1 // BR2,),
        in_specs=[
            pl.BlockSpec((BR2, CH1), lambda i: (i, 0)),
            pl.BlockSpec((1, CH1), lambda i: (0, 0)),
            pl.BlockSpec((1, CH1), lambda i: (0, 0)),
            pl.BlockSpec((1, CH1), lambda i: (0, 0)),
            pl.BlockSpec((1, CH1), lambda i: (0, 0)),
        ],
        out_specs=pl.BlockSpec((BR2, CH1), lambda i: (i, 0)),
        out_shape=jax.ShapeDtypeStruct((B * N1, CH1), f32),
    )(y1, s1, ss1, g1.reshape(1, CH1), beta1.reshape(1, CH1))

    return out.reshape(B, N1, CH1)


# speedup vs baseline: 1.1045x; 1.1045x over previous
import functools

import jax
import jax.numpy as jnp
from jax.experimental import pallas as pl

BQ = 512
BR = 2048
BR2 = 4096


def _knn_mlp0(x1_ref, x2_ref, p1_ref, p2_ref, w0a_ref, w0b_ref, b0_ref,
              y0_ref, s0_ref, ss0_ref):
    b = pl.program_id(0)
    i = pl.program_id(1)
    x1 = x1_ref[0]
    x2t = x2_ref[0].T
    rowsq = jnp.sum(x1 * x1, axis=1, keepdims=True)
    colsq = jnp.sum(x2t * x2t, axis=0, keepdims=True)
    d = rowsq - 2.0 * jnp.dot(x1, x2t, preferred_element_type=jnp.float32)
    d = d + colsq

    inf = jnp.float32(jnp.inf)
    m0 = jnp.min(d, axis=1, keepdims=True)
    d1 = jnp.where(d == m0, inf, d)
    m1 = jnp.min(d1, axis=1, keepdims=True)
    d2 = jnp.where(d1 == m1, inf, d1)
    m2 = jnp.min(d2, axis=1, keepdims=True)

    dists = [jnp.maximum(m, 1e-10) for m in (m0, m1, m2)]
    w = [1.0 / dd for dd in dists]
    norm = jnp.maximum(w[0] + w[1] + w[2], 1e-8)
    wn = [wi / norm for wi in w]

    sel = jnp.where(d == m0, wn[0],
                    jnp.where(d == m1, wn[1],
                              jnp.where(d == m2, wn[2], 0.0)))
    interp = jnp.dot(sel, p2_ref[0], preferred_element_type=jnp.float32)

    y0 = (jnp.dot(p1_ref[0], w0a_ref[...], preferred_element_type=jnp.float32)
          + jnp.dot(interp, w0b_ref[...], preferred_element_type=jnp.float32)
          + b0_ref[...])
    y0_ref[0] = y0

    @pl.when(jnp.logical_and(b == 0, i == 0))
    def _():
        s0_ref[...] = jnp.zeros_like(s0_ref)
        ss0_ref[...] = jnp.zeros_like(ss0_ref)

    s0_ref[...] += jnp.sum(y0, axis=0, keepdims=True)
    ss0_ref[...] += jnp.sum(y0 * y0, axis=0, keepdims=True)


def _mlp1(y0_ref, s0_ref, ss0_ref, g0_ref, beta0_ref, w1t_ref, b1_ref,
          y1_ref, s1_ref, ss1_ref, *, n_total):
    i = pl.program_id(0)
    mean = s0_ref[...] / n_total
    var = ss0_ref[...] / n_total - mean * mean
    scale = g0_ref[...] * jax.lax.rsqrt(var + 1e-5)
    shift = beta0_ref[...] - mean * scale
    z = jnp.maximum(y0_ref[...] * scale + shift, 0.0)
    y1 = jnp.dot(z, w1t_ref[...], preferred_element_type=jnp.float32) + b1_ref[...]
    y1_ref[...] = y1

    @pl.when(i == 0)
    def _():
        s1_ref[...] = jnp.zeros_like(s1_ref)
        ss1_ref[...] = jnp.zeros_like(ss1_ref)

    s1_ref[...] += jnp.sum(y1, axis=0, keepdims=True)
    ss1_ref[...] += jnp.sum(y1 * y1, axis=0, keepdims=True)


def _bn2(y1_ref, s1_ref, ss1_ref, g1_ref, beta1_ref, out_ref, *, n_total):
    mean = s1_ref[...] / n_total
    var = ss1_ref[...] / n_total - mean * mean
    scale = g1_ref[...] * jax.lax.rsqrt(var + 1e-5)
    shift = beta1_ref[...] - mean * scale
    out_ref[...] = jnp.maximum(y1_ref[...] * scale + shift, 0.0)


def kernel(xyz1, xyz2, points1, points2, W0, b0, g0, beta0, W1, b1, g1, beta1):
    B, N1, _ = xyz1.shape
    N2 = xyz2.shape[1]
    C1 = points1.shape[2]
    C2 = points2.shape[2]
    CH0 = W0.shape[0]
    CH1 = W1.shape[0]
    f32 = jnp.float32

    w0a = W0[:, :C1].T
    w0b = W0[:, C1:].T
    w1t = W1.T

    grid_a = (B, N1 // BQ)
    y0, s0, ss0 = pl.pallas_call(
        _knn_mlp0,
        grid=grid_a,
        in_specs=[
            pl.BlockSpec((1, BQ, 3), lambda b, i: (b, i, 0)),
            pl.BlockSpec((1, N2, 3), lambda b, i: (b, 0, 0)),
            pl.BlockSpec((1, BQ, C1), lambda b, i: (b, i, 0)),
            pl.BlockSpec((1, N2, C2), lambda b, i: (b, 0, 0)),
            pl.BlockSpec((C1, CH0), lambda b, i: (0, 0)),
            pl.BlockSpec((C2, CH0), lambda b, i: (0, 0)),
            pl.BlockSpec((1, CH0), lambda b, i: (0, 0)),
        ],
        out_specs=[
            pl.BlockSpec((1, BQ, CH0), lambda b, i: (b, i, 0)),
            pl.BlockSpec((1, CH0), lambda b, i: (0, 0)),
            pl.BlockSpec((1, CH0), lambda b, i: (0, 0)),
        ],
        out_shape=[
            jax.ShapeDtypeStruct((B, N1, CH0), f32),
            jax.ShapeDtypeStruct((1, CH0), f32),
            jax.ShapeDtypeStruct((1, CH0), f32),
        ],
    )(xyz1, xyz2, points1, points2, w0a, w0b, b0.reshape(1, CH0))

    n_total = B * N1
    y0f = y0.reshape(B * N1, CH0)
    y1, s1, ss1 = pl.pallas_call(
        functools.partial(_mlp1, n_total=float(n_total)),
        grid=(B * N1 // BR,),
        in_specs=[
            pl.BlockSpec((BR, CH0), lambda i: (i, 0)),
            pl.BlockSpec((1, CH0), lambda i: (0, 0)),
            pl.BlockSpec((1, CH0), lambda i: (0, 0)),
            pl.BlockSpec((1, CH0), lambda i: (0, 0)),
            pl.BlockSpec((1, CH0), lambda i: (0, 0)),
            pl.BlockSpec((CH0, CH1), lambda i: (0, 0)),
            pl.BlockSpec((1, CH1), lambda i: (0, 0)),
        ],
        out_specs=[
            pl.BlockSpec((BR, CH1), lambda i: (i, 0)),
            pl.BlockSpec((1, CH1), lambda i: (0, 0)),
            pl.BlockSpec((1, CH1), lambda i: (0, 0)),
        ],
        out_shape=[
            jax.ShapeDtypeStruct((B * N1, CH1), f32),
            jax.ShapeDtypeStruct((1, CH1), f32),
            jax.ShapeDtypeStruct((1, CH1), f32),
        ],
    )(y0f, s0, ss0, g0.reshape(1, CH0), beta0.reshape(1, CH0), w1t,
      b1.reshape(1, CH1))

    out = pl.pallas_call(
        functools.partial(_bn2, n_total=float(n_total)),
        grid=(B * N1 // BR2,),
        in_specs=[
            pl.BlockSpec((BR2, CH1), lambda i: (i, 0)),
            pl.BlockSpec((1, CH1), lambda i: (0, 0)),
            pl.BlockSpec((1, CH1), lambda i: (0, 0)),
            pl.BlockSpec((1, CH1), lambda i: (0, 0)),
            pl.BlockSpec((1, CH1), lambda i: (0, 0)),
        ],
        out_specs=pl.BlockSpec((BR2, CH1), lambda i: (i, 0)),
        out_shape=jax.ShapeDtypeStruct((B * N1, CH1), f32),
    )(y1, s1, ss1, g1.reshape(1, CH1), beta1.reshape(1, CH1))

    return out.reshape(B, N1, CH1)


# speedup vs baseline: 1.1665x; 1.0561x over previous
import functools

import jax
import jax.numpy as jnp
from jax.experimental import pallas as pl

BQ = 1024
BR = 2048
BR2 = 4096


def _knn_mlp0(x1_ref, x2_ref, p1_ref, p2_ref, w0a_ref, w0b_ref, b0_ref,
              y0_ref, s0_ref, ss0_ref):
    b = pl.program_id(0)
    i = pl.program_id(1)
    x1 = x1_ref[0]
    x2t = x2_ref[0].T
    rowsq = jnp.sum(x1 * x1, axis=1, keepdims=True)
    colsq = jnp.sum(x2t * x2t, axis=0, keepdims=True)
    d = rowsq - 2.0 * jnp.dot(x1, x2t, preferred_element_type=jnp.float32)
    d = d + colsq

    inf = jnp.float32(jnp.inf)
    m0 = jnp.min(d, axis=1, keepdims=True)
    d1 = jnp.where(d == m0, inf, d)
    m1 = jnp.min(d1, axis=1, keepdims=True)
    d2 = jnp.where(d1 == m1, inf, d1)
    m2 = jnp.min(d2, axis=1, keepdims=True)

    dists = [jnp.maximum(m, 1e-10) for m in (m0, m1, m2)]
    w = [1.0 / dd for dd in dists]
    norm = jnp.maximum(w[0] + w[1] + w[2], 1e-8)
    wn = [wi / norm for wi in w]

    sel = jnp.where(d == m0, wn[0],
                    jnp.where(d == m1, wn[1],
                              jnp.where(d == m2, wn[2], 0.0)))
    interp = jnp.dot(sel, p2_ref[0], preferred_element_type=jnp.float32)

    y0 = (jnp.dot(p1_ref[0], w0a_ref[...], preferred_element_type=jnp.float32)
          + jnp.dot(interp, w0b_ref[...], preferred_element_type=jnp.float32)
          + b0_ref[...])
    y0_ref[0] = y0

    @pl.when(jnp.logical_and(b == 0, i == 0))
    def _():
        s0_ref[...] = jnp.zeros_like(s0_ref)
        ss0_ref[...] = jnp.zeros_like(ss0_ref)

    s0_ref[...] += jnp.sum(y0, axis=0, keepdims=True)
    ss0_ref[...] += jnp.sum(y0 * y0, axis=0, keepdims=True)


def _mlp1(y0_ref, s0_ref, ss0_ref, g0_ref, beta0_ref, w1t_ref, b1_ref,
          y1_ref, s1_ref, ss1_ref, *, n_total):
    i = pl.program_id(0)
    mean = s0_ref[...] / n_total
    var = ss0_ref[...] / n_total - mean * mean
    scale = g0_ref[...] * jax.lax.rsqrt(var + 1e-5)
    shift = beta0_ref[...] - mean * scale
    z = jnp.maximum(y0_ref[...] * scale + shift, 0.0)
    y1 = jnp.dot(z, w1t_ref[...], preferred_element_type=jnp.float32) + b1_ref[...]
    y1_ref[...] = y1

    @pl.when(i == 0)
    def _():
        s1_ref[...] = jnp.zeros_like(s1_ref)
        ss1_ref[...] = jnp.zeros_like(ss1_ref)

    s1_ref[...] += jnp.sum(y1, axis=0, keepdims=True)
    ss1_ref[...] += jnp.sum(y1 * y1, axis=0, keepdims=True)


def _bn2(y1_ref, s1_ref, ss1_ref, g1_ref, beta1_ref, out_ref, *, n_total):
    mean = s1_ref[...] / n_total
    var = ss1_ref[...] / n_total - mean * mean
    scale = g1_ref[...] * jax.lax.rsqrt(var + 1e-5)
    shift = beta1_ref[...] - mean * scale
    out_ref[...] = jnp.maximum(y1_ref[...] * scale + shift, 0.0)


def kernel(xyz1, xyz2, points1, points2, W0, b0, g0, beta0, W1, b1, g1, beta1):
    B, N1, _ = xyz1.shape
    N2 = xyz2.shape[1]
    C1 = points1.shape[2]
    C2 = points2.shape[2]
    CH0 = W0.shape[0]
    CH1 = W1.shape[0]
    f32 = jnp.float32

    w0a = W0[:, :C1].T
    w0b = W0[:, C1:].T
    w1t = W1.T

    grid_a = (B, N1 // BQ)
    y0, s0, ss0 = pl.pallas_call(
        _knn_mlp0,
        grid=grid_a,
        in_specs=[
            pl.BlockSpec((1, BQ, 3), lambda b, i: (b, i, 0)),
            pl.BlockSpec((1, N2, 3), lambda b, i: (b, 0, 0)),
            pl.BlockSpec((1, BQ, C1), lambda b, i: (b, i, 0)),
            pl.BlockSpec((1, N2, C2), lambda b, i: (b, 0, 0)),
            pl.BlockSpec((C1, CH0), lambda b, i: (0, 0)),
            pl.BlockSpec((C2, CH0), lambda b, i: (0, 0)),
            pl.BlockSpec((1, CH0), lambda b, i: (0, 0)),
        ],
        out_specs=[
            pl.BlockSpec((1, BQ, CH0), lambda b, i: (b, i, 0)),
            pl.BlockSpec((1, CH0), lambda b, i: (0, 0)),
            pl.BlockSpec((1, CH0), lambda b, i: (0, 0)),
        ],
        out_shape=[
            jax.ShapeDtypeStruct((B, N1, CH0), f32),
            jax.ShapeDtypeStruct((1, CH0), f32),
            jax.ShapeDtypeStruct((1, CH0), f32),
        ],
    )(xyz1, xyz2, points1, points2, w0a, w0b, b0.reshape(1, CH0))

    n_total = B * N1
    y0f = y0.reshape(B * N1, CH0)
    y1, s1, ss1 = pl.pallas_call(
        functools.partial(_mlp1, n_total=float(n_total)),
        grid=(B * N1 // BR,),
        in_specs=[
            pl.BlockSpec((BR, CH0), lambda i: (i, 0)),
            pl.BlockSpec((1, CH0), lambda i: (0, 0)),
            pl.BlockSpec((1, CH0), lambda i: (0, 0)),
            pl.BlockSpec((1, CH0), lambda i: (0, 0)),
            pl.BlockSpec((1, CH0), lambda i: (0, 0)),
            pl.BlockSpec((CH0, CH1), lambda i: (0, 0)),
            pl.BlockSpec((1, CH1), lambda i: (0, 0)),
        ],
        out_specs=[
            pl.BlockSpec((BR, CH1), lambda i: (i, 0)),
            pl.BlockSpec((1, CH1), lambda i: (0, 0)),
            pl.BlockSpec((1, CH1), lambda i: (0, 0)),
        ],
        out_shape=[
            jax.ShapeDtypeStruct((B * N1, CH1), f32),
            jax.ShapeDtypeStruct((1, CH1), f32),
            jax.ShapeDtypeStruct((1, CH1), f32),
        ],
    )(y0f, s0, ss0, g0.reshape(1, CH0), beta0.reshape(1, CH0), w1t,
      b1.reshape(1, CH1))

    out = pl.pallas_call(
        functools.partial(_bn2, n_total=float(n_total)),
        grid=(B * N1 // BR2,),
        in_specs=[
            pl.BlockSpec((BR2, CH1), lambda i: (i, 0)),
            pl.BlockSpec((1, CH1), lambda i: (0, 0)),
            pl.BlockSpec((1, CH1), lambda i: (0, 0)),
            pl.BlockSpec((1, CH1), lambda i: (0, 0)),
            pl.BlockSpec((1, CH1), lambda i: (0, 0)),
        ],
        out_specs=pl.BlockSpec((BR2, CH1), lambda i: (i, 0)),
        out_shape=jax.ShapeDtypeStruct((B * N1, CH1), f32),
    )(y1, s1, ss1, g1.reshape(1, CH1), beta1.reshape(1, CH1))

    return out.reshape(B, N1, CH1)


# speedup vs baseline: 1.2036x; 1.0318x over previous
import functools

import jax
import jax.numpy as jnp
from jax.experimental import pallas as pl

BQ = 2048
BR = 2048
BR2 = 4096


def _knn_mlp0(x1_ref, x2_ref, p1_ref, p2_ref, w0a_ref, w0b_ref, b0_ref,
              y0_ref, s0_ref, ss0_ref):
    b = pl.program_id(0)
    i = pl.program_id(1)
    x1 = x1_ref[0]
    x2t = x2_ref[0].T
    rowsq = jnp.sum(x1 * x1, axis=1, keepdims=True)
    colsq = jnp.sum(x2t * x2t, axis=0, keepdims=True)
    d = rowsq - 2.0 * jnp.dot(x1, x2t, preferred_element_type=jnp.float32)
    d = d + colsq

    inf = jnp.float32(jnp.inf)
    m0 = jnp.min(d, axis=1, keepdims=True)
    d1 = jnp.where(d == m0, inf, d)
    m1 = jnp.min(d1, axis=1, keepdims=True)
    d2 = jnp.where(d1 == m1, inf, d1)
    m2 = jnp.min(d2, axis=1, keepdims=True)

    dists = [jnp.maximum(m, 1e-10) for m in (m0, m1, m2)]
    w = [1.0 / dd for dd in dists]
    norm = jnp.maximum(w[0] + w[1] + w[2], 1e-8)
    wn = [wi / norm for wi in w]

    sel = jnp.where(d == m0, wn[0],
                    jnp.where(d == m1, wn[1],
                              jnp.where(d == m2, wn[2], 0.0)))
    interp = jnp.dot(sel, p2_ref[0], preferred_element_type=jnp.float32)

    y0 = (jnp.dot(p1_ref[0], w0a_ref[...], preferred_element_type=jnp.float32)
          + jnp.dot(interp, w0b_ref[...], preferred_element_type=jnp.float32)
          + b0_ref[...])
    y0_ref[0] = y0

    @pl.when(jnp.logical_and(b == 0, i == 0))
    def _():
        s0_ref[...] = jnp.zeros_like(s0_ref)
        ss0_ref[...] = jnp.zeros_like(ss0_ref)

    s0_ref[...] += jnp.sum(y0, axis=0, keepdims=True)
    ss0_ref[...] += jnp.sum(y0 * y0, axis=0, keepdims=True)


def _mlp1(y0_ref, s0_ref, ss0_ref, g0_ref, beta0_ref, w1t_ref, b1_ref,
          y1_ref, s1_ref, ss1_ref, *, n_total):
    i = pl.program_id(0)
    mean = s0_ref[...] / n_total
    var = ss0_ref[...] / n_total - mean * mean
    scale = g0_ref[...] * jax.lax.rsqrt(var + 1e-5)
    shift = beta0_ref[...] - mean * scale
    z = jnp.maximum(y0_ref[...] * scale + shift, 0.0)
    y1 = jnp.dot(z, w1t_ref[...], preferred_element_type=jnp.float32) + b1_ref[...]
    y1_ref[...] = y1

    @pl.when(i == 0)
    def _():
        s1_ref[...] = jnp.zeros_like(s1_ref)
        ss1_ref[...] = jnp.zeros_like(ss1_ref)

    s1_ref[...] += jnp.sum(y1, axis=0, keepdims=True)
    ss1_ref[...] += jnp.sum(y1 * y1, axis=0, keepdims=True)


def _bn2(y1_ref, s1_ref, ss1_ref, g1_ref, beta1_ref, out_ref, *, n_total):
    mean = s1_ref[...] / n_total
    var = ss1_ref[...] / n_total - mean * mean
    scale = g1_ref[...] * jax.lax.rsqrt(var + 1e-5)
    shift = beta1_ref[...] - mean * scale
    out_ref[...] = jnp.maximum(y1_ref[...] * scale + shift, 0.0)


def kernel(xyz1, xyz2, points1, points2, W0, b0, g0, beta0, W1, b1, g1, beta1):
    B, N1, _ = xyz1.shape
    N2 = xyz2.shape[1]
    C1 = points1.shape[2]
    C2 = points2.shape[2]
    CH0 = W0.shape[0]
    CH1 = W1.shape[0]
    f32 = jnp.float32

    w0a = W0[:, :C1].T
    w0b = W0[:, C1:].T
    w1t = W1.T

    grid_a = (B, N1 // BQ)
    y0, s0, ss0 = pl.pallas_call(
        _knn_mlp0,
        grid=grid_a,
        in_specs=[
            pl.BlockSpec((1, BQ, 3), lambda b, i: (b, i, 0)),
            pl.BlockSpec((1, N2, 3), lambda b, i: (b, 0, 0)),
            pl.BlockSpec((1, BQ, C1), lambda b, i: (b, i, 0)),
            pl.BlockSpec((1, N2, C2), lambda b, i: (b, 0, 0)),
            pl.BlockSpec((C1, CH0), lambda b, i: (0, 0)),
            pl.BlockSpec((C2, CH0), lambda b, i: (0, 0)),
            pl.BlockSpec((1, CH0), lambda b, i: (0, 0)),
        ],
        out_specs=[
            pl.BlockSpec((1, BQ, CH0), lambda b, i: (b, i, 0)),
            pl.BlockSpec((1, CH0), lambda b, i: (0, 0)),
            pl.BlockSpec((1, CH0), lambda b, i: (0, 0)),
        ],
        out_shape=[
            jax.ShapeDtypeStruct((B, N1, CH0), f32),
            jax.ShapeDtypeStruct((1, CH0), f32),
            jax.ShapeDtypeStruct((1, CH0), f32),
        ],
    )(xyz1, xyz2, points1, points2, w0a, w0b, b0.reshape(1, CH0))

    n_total = B * N1
    y0f = y0.reshape(B * N1, CH0)
    y1, s1, ss1 = pl.pallas_call(
        functools.partial(_mlp1, n_total=float(n_total)),
        grid=(B * N1 // BR,),
        in_specs=[
            pl.BlockSpec((BR, CH0), lambda i: (i, 0)),
            pl.BlockSpec((1, CH0), lambda i: (0, 0)),
            pl.BlockSpec((1, CH0), lambda i: (0, 0)),
            pl.BlockSpec((1, CH0), lambda i: (0, 0)),
            pl.BlockSpec((1, CH0), lambda i: (0, 0)),
            pl.BlockSpec((CH0, CH1), lambda i: (0, 0)),
            pl.BlockSpec((1, CH1), lambda i: (0, 0)),
        ],
        out_specs=[
            pl.BlockSpec((BR, CH1), lambda i: (i, 0)),
            pl.BlockSpec((1, CH1), lambda i: (0, 0)),
            pl.BlockSpec((1, CH1), lambda i: (0, 0)),
        ],
        out_shape=[
            jax.ShapeDtypeStruct((B * N1, CH1), f32),
            jax.ShapeDtypeStruct((1, CH1), f32),
            jax.ShapeDtypeStruct((1, CH1), f32),
        ],
    )(y0f, s0, ss0, g0.reshape(1, CH0), beta0.reshape(1, CH0), w1t,
      b1.reshape(1, CH1))

    out = pl.pallas_call(
        functools.partial(_bn2, n_total=float(n_total)),
        grid=(B * N1 // BR2,),
        in_specs=[
            pl.BlockSpec((BR2, CH1), lambda i: (i, 0)),
            pl.BlockSpec((1, CH1), lambda i: (0, 0)),
            pl.BlockSpec((1, CH1), lambda i: (0, 0)),
            pl.BlockSpec((1, CH1), lambda i: (0, 0)),
            pl.BlockSpec((1, CH1), lambda i: (0, 0)),
        ],
        out_specs=pl.BlockSpec((BR2, CH1), lambda i: (i, 0)),
        out_shape=jax.ShapeDtypeStruct((B * N1, CH1), f32),
    )(y1, s1, ss1, g1.reshape(1, CH1), beta1.reshape(1, CH1))

    return out.reshape(B, N1, CH1)
